# baseline (speedup 1.0000x reference)
import functools

import jax
import jax.numpy as jnp
from jax import lax
from jax.experimental import pallas as pl
from jax.experimental.pallas import tpu as pltpu
from jax.experimental.pallas import tpu_sc as plsc

B, F, N, K = 4, 16, 16384, 16
C_OUT = 32
EPS = 1e-5

NC, NS = 2, 16
NW = NC * NS
BN = B * N
PTS_PER_B_W = N // NW
PCH = 64
NCH = PTS_PER_B_W // PCH
GCH = B * NCH
RPC = PCH * K
GSZ = 128
NG = RPC // GSZ
NB = 2048
NBLK = N // NB


NP4 = NB // 4


def _prep_body(x_ref, wa_ref, wd_ref, u_ref, v_ref, vs_ref, vq_ref):
    x4 = x_ref[0]
    u = jnp.dot(x4, wa_ref[...], preferred_element_type=jnp.float32)
    v = jnp.dot(x4, wd_ref[...], preferred_element_type=jnp.float32)
    u_ref[...] = u
    v_ref[...] = v

    @pl.when((pl.program_id(0) == 0) & (pl.program_id(1) == 0))
    def _():
        vs_ref[...] = jnp.zeros_like(vs_ref)
        vq_ref[...] = jnp.zeros_like(vq_ref)

    vs_ref[...] += jnp.sum(v, axis=0)
    vq_ref[...] += jnp.sum(v * v, axis=0)


def _prep(x4, w4a, w4d):
    return pl.pallas_call(
        _prep_body,
        grid=(B, NBLK),
        in_specs=[
            pl.BlockSpec((1, NP4, 4 * F), lambda b, j: (b, j, 0)),
            pl.BlockSpec((4 * F, 128), lambda b, j: (0, 0)),
            pl.BlockSpec((4 * F, 128), lambda b, j: (0, 0)),
        ],
        out_specs=[
            pl.BlockSpec((NP4, 128), lambda b, j: (b * NBLK + j, 0)),
            pl.BlockSpec((NP4, 128), lambda b, j: (b * NBLK + j, 0)),
            pl.BlockSpec((128,), lambda b, j: (0,)),
            pl.BlockSpec((128,), lambda b, j: (0,)),
        ],
        out_shape=[
            jax.ShapeDtypeStruct((BN // 4, 128), jnp.float32),
            jax.ShapeDtypeStruct((BN // 4, 128), jnp.float32),
            jax.ShapeDtypeStruct((128,), jnp.float32),
            jax.ShapeDtypeStruct((128,), jnp.float32),
        ],
    )(x4, w4a, w4d)


ICH = 256
ICN = (BN // NW) // ICH


def _idx_body(idx_hbm, out_hbm, s0, s1, d0, d1, cs0, cs1, ds0, ds1):
    wid = lax.axis_index("s") * NC + lax.axis_index("c")
    p0 = wid * (BN // NW)
    b = p0 // N
    boff = b * N
    srcs, dsts = (s0, s1), (d0, d1)
    csems, dsems = (cs0, cs1), (ds0, ds1)

    def fire(c, j):
        n0 = pl.multiple_of((p0 + c * ICH) % N, ICH)
        pltpu.async_copy(idx_hbm.at[b, pl.ds(n0, ICH)], srcs[j], csems[j])

    def wait_fire(c, j):
        n0 = pl.multiple_of((p0 + c * ICH) % N, ICH)
        pltpu.make_async_copy(
            idx_hbm.at[b, pl.ds(n0, ICH)], srcs[j], csems[j]).wait()

    def run(c, j):
        wait_fire(c, j)
        src, dst = srcs[j], dsts[j]

        @pl.when(c >= 2)
        def _():
            pltpu.make_async_copy(
                dst, out_hbm.at[pl.ds(0, ICH * K // 128)], dsems[j]).wait()

        def mv(i, _):
            dst[i // 8, pl.ds((i % 8) * K, K)] = src[i, :] + boff
            return 0
        lax.fori_loop(0, ICH, mv, 0)
        r0 = pl.multiple_of((p0 + c * ICH) * K // 128, ICH * K // 128)
        pltpu.async_copy(dst, out_hbm.at[pl.ds(r0, ICH * K // 128)], dsems[j])

    fire(0, 0)
    fire(1, 1)

    def loop(c2, _):
        c = c2 * 2
        run(c, 0)

        @pl.when(c + 2 < ICN)
        def _():
            fire(c + 2, 0)
        run(c + 1, 1)

        @pl.when(c + 3 < ICN)
        def _():
            fire(c + 3, 1)
        return 0

    lax.fori_loop(0, ICN // 2, loop, 0)
    pltpu.make_async_copy(
        d0, out_hbm.at[pl.ds(0, ICH * K // 128)], ds0).wait()
    pltpu.make_async_copy(
        d1, out_hbm.at[pl.ds(0, ICH * K // 128)], ds1).wait()


def _idx_relayout(idx):
    mesh = plsc.VectorSubcoreMesh(core_axis_name="c", subcore_axis_name="s")
    kfn = functools.partial(
        pl.kernel, mesh=mesh,
        compiler_params=pltpu.CompilerParams(
            use_tc_tiling_on_sc=True, needs_layout_passes=False),
        out_type=jax.ShapeDtypeStruct((BN * K // 128, 128), jnp.int32),
        scratch_types=(
            [pltpu.VMEM((ICH, K), jnp.int32) for _ in range(2)]
            + [pltpu.VMEM((ICH * K // 128, 128), jnp.int32) for _ in range(2)]
            + [pltpu.SemaphoreType.DMA for _ in range(4)]
        ),
    )(_idx_body)
    return kfn(idx)


UCH = 512
UCN = (BN // NW) // UCH


def _u_body(u128_hbm, out_hbm, s0, s1, d0, d1, cs0, cs1, ds0, ds1):
    wid = lax.axis_index("s") * NC + lax.axis_index("c")
    p0 = wid * (BN // NW)
    srcs, dsts = (s0, s1), (d0, d1)
    csems, dsems = (cs0, cs1), (ds0, ds1)

    def fire(c, j):
        r0 = (p0 + c * UCH) // 4
        pltpu.async_copy(u128_hbm.at[pl.ds(r0, UCH // 4)], srcs[j], csems[j])

    def run(c, j):
        r0 = (p0 + c * UCH) // 4
        src, dst = srcs[j], dsts[j]
        pltpu.make_async_copy(
            u128_hbm.at[pl.ds(r0, UCH // 4)], src, csems[j]).wait()

        @pl.when(c >= 2)
        def _():
            pltpu.make_async_copy(
                dst, out_hbm.at[pl.ds(0, UCH)], dsems[j]).wait()

        def mv(i, _):
            q = (i % 4) * 32
            dst[i, pl.ds(0, 16)] = src[i // 4, pl.ds(q, 16)]
            dst[i, pl.ds(16, 16)] = src[i // 4, pl.ds(q + 16, 16)]
            return 0
        lax.fori_loop(0, UCH, mv, 0)
        pltpu.async_copy(dst, out_hbm.at[pl.ds(p0 + c * UCH, UCH)], dsems[j])

    fire(0, 0)
    fire(1, 1)

    def loop(c2, _):
        c = c2 * 2
        run(c, 0)

        @pl.when(c + 2 < UCN)
        def _():
            fire(c + 2, 0)
        run(c + 1, 1)

        @pl.when(c + 3 < UCN)
        def _():
            fire(c + 3, 1)
        return 0

    lax.fori_loop(0, UCN // 2, loop, 0)
    pltpu.make_async_copy(d0, out_hbm.at[pl.ds(0, UCH)], ds0).wait()
    pltpu.make_async_copy(d1, out_hbm.at[pl.ds(0, UCH)], ds1).wait()


def _u_relayout(u128):
    mesh = plsc.VectorSubcoreMesh(core_axis_name="c", subcore_axis_name="s")
    kfn = functools.partial(
        pl.kernel, mesh=mesh,
        compiler_params=pltpu.CompilerParams(
            use_tc_tiling_on_sc=False, needs_layout_passes=False),
        out_type=jax.ShapeDtypeStruct((BN, C_OUT), jnp.float32),
        scratch_types=(
            [pltpu.VMEM((UCH // 4, 128), jnp.float32) for _ in range(2)]
            + [pltpu.VMEM((UCH, C_OUT), jnp.float32) for _ in range(2)]
            + [pltpu.SemaphoreType.DMA for _ in range(4)]
        ),
    )(_u_body)
    return kfn(u128)


def _sc_body(u_hbm, idx_hbm, v_hbm, a_hbm, parts_hbm,
             i0, i1, i2, i3, v0, v1, v2, v3,
             rows0, rows1, ab0, ab1, sbuf,
             is0, is1, is2, is3, vs0, vs1, vs2, vs3,
             gs0, gs1, as0, as1):
    wid = lax.axis_index("s") * NC + lax.axis_index("c")
    base = wid * PTS_PER_B_W
    zero = jnp.zeros((16,), jnp.float32)
    carry = (zero, zero, zero, zero, zero, zero)
    lane = lax.iota(jnp.int32, 16)

    idxs = (i0, i1, i2, i3)
    vbufs = (v0, v1, v2, v3)
    isems = (is0, is1, is2, is3)
    vsems = (vs0, vs1, vs2, vs3)
    rowss = (rows0, rows1)
    abufs = (ab0, ab1)
    gsems = (gs0, gs1)
    asems = (as0, as1)

    def pt0_of(g):
        return (g // NCH) * N + base + (g % NCH) * PCH

    def a_dst(g):
        b = g // NCH
        n0 = base + (g % NCH) * PCH
        return a_hbm.at[pl.ds(b * C_OUT, C_OUT), pl.ds(n0, PCH)]

    def fire_in(g, j):
        pt0 = pt0_of(g)
        pltpu.async_copy(idx_hbm.at[pl.ds(pt0 // 8, NG)], idxs[j], isems[j])
        pltpu.async_copy(v_hbm.at[pl.ds(pt0 // 4, PCH // 4)], vbufs[j], vsems[j])

    def arm(g, j, r):
        pltpu.make_async_copy(
            idx_hbm.at[pl.ds(pt0_of(g) // 8, NG)], idxs[j], isems[j]).wait()
        for q in range(NG):
            pltpu.async_copy(
                u_hbm.at[idxs[j].at[q, :]],
                rowss[r].at[pl.ds(q * GSZ, GSZ)], gsems[r])

    def work(g, j, r, carry):
        pt0 = pt0_of(g)
        rows, vbuf, abuf = rowss[r], vbufs[j], abufs[r]
        for q in range(NG):
            pltpu.make_async_copy(
                u_hbm.at[idxs[j].at[q, :]],
                rows.at[pl.ds(q * GSZ, GSZ)], gsems[r]).wait()
        pltpu.make_async_copy(
            v_hbm.at[pl.ds(pt0 // 4, PCH // 4)], vbuf, vsems[j]).wait()

        @pl.when(g >= 2)
        def _():
            pltpu.make_async_copy(abuf, a_dst(g), asems[r]).wait()

        def pt_body(p, c):
            s1a, s1b, s2a, s2b, s3a, s3b = c
            r0 = p * K
            m0 = rows[r0, pl.ds(0, 16)]
            m1 = rows[r0, pl.ds(16, 16)]
            sa, sb = m0, m1
            qa, qb = m0 * m0, m1 * m1
            for k in range(1, K):
                ra = rows[r0 + k, pl.ds(0, 16)]
                rb = rows[r0 + k, pl.ds(16, 16)]
                m0 = jnp.maximum(m0, ra)
                m1 = jnp.maximum(m1, rb)
                sa = sa + ra
                sb = sb + rb
                qa = qa + ra * ra
                qb = qb + rb * rb
            va = vbuf[p // 4, pl.ds((p % 4) * 32, 16)]
            vb = vbuf[p // 4, pl.ds((p % 4) * 32 + 16, 16)]
            pcol = jnp.broadcast_to(p, (16,))
            plsc.store_scatter(abuf, [lane, pcol], m0 + va)
            plsc.store_scatter(abuf, [lane + 16, pcol], m1 + vb)
            return (s1a + sa, s1b + sb, s2a + qa, s2b + qb,
                    s3a + sa * va, s3b + sb * vb)

        carry = lax.fori_loop(0, PCH, pt_body, carry)
        pltpu.async_copy(abuf, a_dst(g), asems[r])
        return carry

    for g in range(4):
        fire_in(g, g)
    arm(0, 0, 0)

    def quad_body(c4, carry):
        g0 = c4 * 4
        for s in range(4):
            g = g0 + s
            j = s
            r = s % 2

            if s < 3:
                carry_arm = (g + 1, (s + 1), (s + 1) % 2)
            else:
                carry_arm = (g + 1, 0, 0)
            na_g, na_j, na_r = carry_arm

            @pl.when(na_g < GCH)
            def _(na_g=na_g, na_j=na_j, na_r=na_r):
                arm(na_g, na_j, na_r)
            carry = work(g, j, r, carry)

            @pl.when(g + 4 < GCH)
            def _(g=g, j=j):
                fire_in(g + 4, j)
        return carry

    carry = lax.fori_loop(0, GCH // 4, quad_body, carry)

    pltpu.make_async_copy(ab0, a_dst(GCH - 2), as0).wait()
    pltpu.make_async_copy(ab1, a_dst(GCH - 1), as1).wait()

    for i in range(6):
        sbuf[i, :] = carry[i]
    pltpu.sync_copy(sbuf, parts_hbm.at[wid])


def _sc_gather(u_flat, idx_flat, v_flat):
    mesh = plsc.VectorSubcoreMesh(core_axis_name="c", subcore_axis_name="s")
    kfn = functools.partial(
        pl.kernel, mesh=mesh,
        compiler_params=pltpu.CompilerParams(
            use_tc_tiling_on_sc=False, needs_layout_passes=False),
        out_type=(
            jax.ShapeDtypeStruct((B * C_OUT, N), jnp.float32),
            jax.ShapeDtypeStruct((NW, 6, 16), jnp.float32),
        ),
        scratch_types=(
            [pltpu.VMEM((NG, 128), jnp.int32) for _ in range(4)]
            + [pltpu.VMEM((PCH // 4, 128), jnp.float32) for _ in range(4)]
            + [pltpu.VMEM((RPC, C_OUT), jnp.float32) for _ in range(2)]
            + [pltpu.VMEM((C_OUT, PCH), jnp.float32) for _ in range(2)]
            + [pltpu.VMEM((6, 16), jnp.float32)]
            + [pltpu.SemaphoreType.DMA for _ in range(12)]
        ),
    )(_sc_body)
    return kfn(u_flat, idx_flat, v_flat)


def _final_body(a_ref, s_ref, t_ref, o_ref):
    a = a_ref[...]
    y = a * s_ref[...][:, None] + t_ref[...][:, None]
    y = jnp.where(y >= 0, y, 0.2 * y)
    o_ref[0] = y


def _final(a_t, scale, shift):
    return pl.pallas_call(
        _final_body,
        grid=(B, NBLK),
        in_specs=[
            pl.BlockSpec((C_OUT, NB), lambda b, j: (b, j)),
            pl.BlockSpec((C_OUT,), lambda b, j: (0,)),
            pl.BlockSpec((C_OUT,), lambda b, j: (0,)),
        ],
        out_specs=pl.BlockSpec((1, C_OUT, NB), lambda b, j: (b, 0, j)),
        out_shape=jax.ShapeDtypeStruct((B, C_OUT, N), jnp.float32),
    )(a_t, scale, shift)


def kernel(x, fixed_knn_graph, W1, g1, b1):
    wa_t = W1[:, :F].T
    wd_t = (W1[:, F:] - W1[:, :F]).T
    eye4 = jnp.eye(4, dtype=jnp.float32)
    w4a = jnp.kron(eye4, wa_t)
    w4d = jnp.kron(eye4, wd_t)
    x4 = x.transpose(0, 2, 1).reshape(B, N // 4, 4 * F)
    u128, v128, vs128, vq128 = _prep(x4, w4a, w4d)
    idx128 = _idx_relayout(fixed_knn_graph)

    a_t, parts = _sc_gather(_u_relayout(u128), idx128, v128)

    s = jnp.sum(parts, axis=0)
    usum = s[0:2].reshape(C_OUT)
    usq = s[2:4].reshape(C_OUT)
    ucross = s[4:6].reshape(C_OUT)
    vsum = vs128.reshape(4, C_OUT).sum(axis=0)
    vsq = vq128.reshape(4, C_OUT).sum(axis=0)
    cnt = float(B * N * K)
    mean = (usum + K * vsum) / cnt
    ey2 = (usq + 2.0 * ucross + K * vsq) / cnt
    var = ey2 - mean * mean
    scale = g1 * lax.rsqrt(var + EPS)
    shift = b1 - scale * mean

    return _final(a_t, scale, shift)


# speedup vs baseline: 1.0939x; 1.0939x over previous
import functools

import jax
import jax.numpy as jnp
from jax import lax
from jax.experimental import pallas as pl
from jax.experimental.pallas import tpu as pltpu
from jax.experimental.pallas import tpu_sc as plsc

B, F, N, K = 4, 16, 16384, 16
C_OUT = 32
EPS = 1e-5

NC, NS = 2, 16
NW = NC * NS
BN = B * N
PTS_PER_B_W = N // NW
PCH = 64
NCH = PTS_PER_B_W // PCH
GCH = B * NCH
RPC = PCH * K
GSZ = 128
NG = RPC // GSZ
NB = 2048
NBLK = N // NB


NP4 = NB // 4


def _prep_body(x_ref, wa_ref, wd_ref, u_ref, v_ref, vs_ref, vq_ref):
    x4 = x_ref[0]
    u = jnp.dot(x4, wa_ref[...], preferred_element_type=jnp.float32)
    v = jnp.dot(x4, wd_ref[...], preferred_element_type=jnp.float32)
    u_ref[...] = u
    v_ref[...] = v

    @pl.when((pl.program_id(0) == 0) & (pl.program_id(1) == 0))
    def _():
        vs_ref[...] = jnp.zeros_like(vs_ref)
        vq_ref[...] = jnp.zeros_like(vq_ref)

    vs_ref[...] += jnp.sum(v, axis=0)
    vq_ref[...] += jnp.sum(v * v, axis=0)


def _prep(x4, w4a, w4d):
    return pl.pallas_call(
        _prep_body,
        grid=(B, NBLK),
        in_specs=[
            pl.BlockSpec((1, NP4, 4 * F), lambda b, j: (b, j, 0)),
            pl.BlockSpec((4 * F, 128), lambda b, j: (0, 0)),
            pl.BlockSpec((4 * F, 128), lambda b, j: (0, 0)),
        ],
        out_specs=[
            pl.BlockSpec((NP4, 128), lambda b, j: (b * NBLK + j, 0)),
            pl.BlockSpec((NP4, 128), lambda b, j: (b * NBLK + j, 0)),
            pl.BlockSpec((128,), lambda b, j: (0,)),
            pl.BlockSpec((128,), lambda b, j: (0,)),
        ],
        out_shape=[
            jax.ShapeDtypeStruct((BN // 4, 128), jnp.float32),
            jax.ShapeDtypeStruct((BN // 4, 128), jnp.float32),
            jax.ShapeDtypeStruct((128,), jnp.float32),
            jax.ShapeDtypeStruct((128,), jnp.float32),
        ],
    )(x4, w4a, w4d)


ICH = 256
ICN = (BN // NW) // ICH


def _idx_body(idx_hbm, out_hbm, s0, s1, d0, d1, cs0, cs1, ds0, ds1):
    wid = lax.axis_index("s") * NC + lax.axis_index("c")
    p0 = wid * (BN // NW)
    b = p0 // N
    boff = b * N
    srcs, dsts = (s0, s1), (d0, d1)
    csems, dsems = (cs0, cs1), (ds0, ds1)

    def fire(c, j):
        n0 = pl.multiple_of((p0 + c * ICH) % N, ICH)
        pltpu.async_copy(idx_hbm.at[b, pl.ds(n0, ICH)], srcs[j], csems[j])

    def wait_fire(c, j):
        n0 = pl.multiple_of((p0 + c * ICH) % N, ICH)
        pltpu.make_async_copy(
            idx_hbm.at[b, pl.ds(n0, ICH)], srcs[j], csems[j]).wait()

    def run(c, j):
        wait_fire(c, j)
        src, dst = srcs[j], dsts[j]

        @pl.when(c >= 2)
        def _():
            pltpu.make_async_copy(
                dst, out_hbm.at[pl.ds(0, ICH * K // 128)], dsems[j]).wait()

        def mv(i, _):
            dst[i // 8, pl.ds((i % 8) * K, K)] = src[i, :] + boff
            return 0
        lax.fori_loop(0, ICH, mv, 0)
        r0 = pl.multiple_of((p0 + c * ICH) * K // 128, ICH * K // 128)
        pltpu.async_copy(dst, out_hbm.at[pl.ds(r0, ICH * K // 128)], dsems[j])

    fire(0, 0)
    fire(1, 1)

    def loop(c2, _):
        c = c2 * 2
        run(c, 0)

        @pl.when(c + 2 < ICN)
        def _():
            fire(c + 2, 0)
        run(c + 1, 1)

        @pl.when(c + 3 < ICN)
        def _():
            fire(c + 3, 1)
        return 0

    lax.fori_loop(0, ICN // 2, loop, 0)
    pltpu.make_async_copy(
        d0, out_hbm.at[pl.ds(0, ICH * K // 128)], ds0).wait()
    pltpu.make_async_copy(
        d1, out_hbm.at[pl.ds(0, ICH * K // 128)], ds1).wait()


def _idx_relayout(idx):
    mesh = plsc.VectorSubcoreMesh(core_axis_name="c", subcore_axis_name="s")
    kfn = functools.partial(
        pl.kernel, mesh=mesh,
        compiler_params=pltpu.CompilerParams(
            use_tc_tiling_on_sc=True, needs_layout_passes=False),
        out_type=jax.ShapeDtypeStruct((BN * K // 128, 128), jnp.int32),
        scratch_types=(
            [pltpu.VMEM((ICH, K), jnp.int32) for _ in range(2)]
            + [pltpu.VMEM((ICH * K // 128, 128), jnp.int32) for _ in range(2)]
            + [pltpu.SemaphoreType.DMA for _ in range(4)]
        ),
    )(_idx_body)
    return kfn(idx)


def _sc_body(u_hbm, idx_hbm, v_hbm, a_hbm, parts_hbm,
             i0, i1, i2, i3, v0, v1, v2, v3,
             rows0, rows1, ab0, ab1, sbuf,
             is0, is1, is2, is3, vs0, vs1, vs2, vs3,
             gs0, gs1, as0, as1):
    wid = lax.axis_index("s") * NC + lax.axis_index("c")
    base = wid * PTS_PER_B_W
    zero = jnp.zeros((16,), jnp.float32)
    carry = (zero, zero, zero, zero, zero, zero)
    lane = lax.iota(jnp.int32, 16)

    idxs = (i0, i1, i2, i3)
    vbufs = (v0, v1, v2, v3)
    isems = (is0, is1, is2, is3)
    vsems = (vs0, vs1, vs2, vs3)
    rowss = (rows0, rows1)
    abufs = (ab0, ab1)
    gsems = (gs0, gs1)
    asems = (as0, as1)

    def pt0_of(g):
        return (g // NCH) * N + base + (g % NCH) * PCH

    def a_dst(g):
        b = g // NCH
        n0 = base + (g % NCH) * PCH
        return a_hbm.at[pl.ds(b * C_OUT, C_OUT), pl.ds(n0, PCH)]

    def fire_in(g, j):
        pt0 = pt0_of(g)
        pltpu.async_copy(idx_hbm.at[pl.ds(pt0 // 8, NG)], idxs[j], isems[j])
        pltpu.async_copy(v_hbm.at[pl.ds(pt0 // 4, PCH // 4)], vbufs[j], vsems[j])

    def arm(g, j, r):
        pltpu.make_async_copy(
            idx_hbm.at[pl.ds(pt0_of(g) // 8, NG)], idxs[j], isems[j]).wait()
        for q in range(NG):
            pltpu.async_copy(
                u_hbm.at[idxs[j].at[q, :]],
                rowss[r].at[pl.ds(q * GSZ, GSZ)], gsems[r])

    def work(g, j, r, carry):
        pt0 = pt0_of(g)
        rows, vbuf, abuf = rowss[r], vbufs[j], abufs[r]
        for q in range(NG):
            pltpu.make_async_copy(
                u_hbm.at[idxs[j].at[q, :]],
                rows.at[pl.ds(q * GSZ, GSZ)], gsems[r]).wait()
        pltpu.make_async_copy(
            v_hbm.at[pl.ds(pt0 // 4, PCH // 4)], vbuf, vsems[j]).wait()

        @pl.when(g >= 2)
        def _():
            pltpu.make_async_copy(abuf, a_dst(g), asems[r]).wait()

        def pt_body(p, c):
            s1a, s1b, s2a, s2b, s3a, s3b = c
            r0 = p * K
            m0 = rows[r0, pl.ds(0, 16)]
            m1 = rows[r0, pl.ds(16, 16)]
            sa, sb = m0, m1
            qa, qb = m0 * m0, m1 * m1
            for k in range(1, K):
                ra = rows[r0 + k, pl.ds(0, 16)]
                rb = rows[r0 + k, pl.ds(16, 16)]
                m0 = jnp.maximum(m0, ra)
                m1 = jnp.maximum(m1, rb)
                sa = sa + ra
                sb = sb + rb
                qa = qa + ra * ra
                qb = qb + rb * rb
            va = vbuf[p // 4, pl.ds((p % 4) * 32, 16)]
            vb = vbuf[p // 4, pl.ds((p % 4) * 32 + 16, 16)]
            pcol = jnp.broadcast_to(p, (16,))
            plsc.store_scatter(abuf, [lane, pcol], m0 + va)
            plsc.store_scatter(abuf, [lane + 16, pcol], m1 + vb)
            return (s1a + sa, s1b + sb, s2a + qa, s2b + qb,
                    s3a + sa * va, s3b + sb * vb)

        carry = lax.fori_loop(0, PCH, pt_body, carry)
        pltpu.async_copy(abuf, a_dst(g), asems[r])
        return carry

    for g in range(4):
        fire_in(g, g)
    arm(0, 0, 0)

    def quad_body(c4, carry):
        g0 = c4 * 4
        for s in range(4):
            g = g0 + s
            j = s
            r = s % 2

            if s < 3:
                carry_arm = (g + 1, (s + 1), (s + 1) % 2)
            else:
                carry_arm = (g + 1, 0, 0)
            na_g, na_j, na_r = carry_arm

            @pl.when(na_g < GCH)
            def _(na_g=na_g, na_j=na_j, na_r=na_r):
                arm(na_g, na_j, na_r)
            carry = work(g, j, r, carry)

            @pl.when(g + 4 < GCH)
            def _(g=g, j=j):
                fire_in(g + 4, j)
        return carry

    carry = lax.fori_loop(0, GCH // 4, quad_body, carry)

    pltpu.make_async_copy(ab0, a_dst(GCH - 2), as0).wait()
    pltpu.make_async_copy(ab1, a_dst(GCH - 1), as1).wait()

    for i in range(6):
        sbuf[i, :] = carry[i]
    pltpu.sync_copy(sbuf, parts_hbm.at[wid])


def _sc_gather(u_flat, idx_flat, v_flat):
    mesh = plsc.VectorSubcoreMesh(core_axis_name="c", subcore_axis_name="s")
    kfn = functools.partial(
        pl.kernel, mesh=mesh,
        compiler_params=pltpu.CompilerParams(
            use_tc_tiling_on_sc=False, needs_layout_passes=False),
        out_type=(
            jax.ShapeDtypeStruct((B * C_OUT, N), jnp.float32),
            jax.ShapeDtypeStruct((NW, 6, 16), jnp.float32),
        ),
        scratch_types=(
            [pltpu.VMEM((NG, 128), jnp.int32) for _ in range(4)]
            + [pltpu.VMEM((PCH // 4, 128), jnp.float32) for _ in range(4)]
            + [pltpu.VMEM((RPC, C_OUT), jnp.float32) for _ in range(2)]
            + [pltpu.VMEM((C_OUT, PCH), jnp.float32) for _ in range(2)]
            + [pltpu.VMEM((6, 16), jnp.float32)]
            + [pltpu.SemaphoreType.DMA for _ in range(12)]
        ),
    )(_sc_body)
    return kfn(u_flat, idx_flat, v_flat)


def _final_body(a_ref, s_ref, t_ref, o_ref):
    a = a_ref[...]
    y = a * s_ref[...][:, None] + t_ref[...][:, None]
    y = jnp.where(y >= 0, y, 0.2 * y)
    o_ref[0] = y


def _final(a_t, scale, shift):
    return pl.pallas_call(
        _final_body,
        grid=(B, NBLK),
        in_specs=[
            pl.BlockSpec((C_OUT, NB), lambda b, j: (b, j)),
            pl.BlockSpec((C_OUT,), lambda b, j: (0,)),
            pl.BlockSpec((C_OUT,), lambda b, j: (0,)),
        ],
        out_specs=pl.BlockSpec((1, C_OUT, NB), lambda b, j: (b, 0, j)),
        out_shape=jax.ShapeDtypeStruct((B, C_OUT, N), jnp.float32),
    )(a_t, scale, shift)


def kernel(x, fixed_knn_graph, W1, g1, b1):
    wa_t = W1[:, :F].T
    wd_t = (W1[:, F:] - W1[:, :F]).T
    eye4 = jnp.eye(4, dtype=jnp.float32)
    w4a = jnp.kron(eye4, wa_t)
    w4d = jnp.kron(eye4, wd_t)
    x4 = x.transpose(0, 2, 1).reshape(B, N // 4, 4 * F)
    u128, v128, vs128, vq128 = _prep(x4, w4a, w4d)
    idx128 = _idx_relayout(fixed_knn_graph)

    a_t, parts = _sc_gather(u128.reshape(BN, C_OUT), idx128, v128)

    s = jnp.sum(parts, axis=0)
    usum = s[0:2].reshape(C_OUT)
    usq = s[2:4].reshape(C_OUT)
    ucross = s[4:6].reshape(C_OUT)
    vsum = vs128.reshape(4, C_OUT).sum(axis=0)
    vsq = vq128.reshape(4, C_OUT).sum(axis=0)
    cnt = float(B * N * K)
    mean = (usum + K * vsum) / cnt
    ey2 = (usq + 2.0 * ucross + K * vsq) / cnt
    var = ey2 - mean * mean
    scale = g1 * lax.rsqrt(var + EPS)
    shift = b1 - scale * mean

    return _final(a_t, scale, shift)
